# baseline (speedup 1.0000x reference)
import functools

import jax
import jax.numpy as jnp
from jax import lax
from jax.experimental import pallas as pl
from jax.experimental.pallas import tpu as pltpu
from jax.experimental.pallas import tpu_sc as plsc

N = 50000
M = 16
F_ATOM = 64
F_NBR = 41
F_OUT = 128
EDGES = N * M
EPS = 1e-5

_B = 400
_C = 1000
_HB = _B * M // 2


def _sigmoid(x):
    return 1.0 / (1.0 + jnp.exp(-x))


def _softplus(x):
    return jnp.maximum(x, 0.0) + jnp.log(1.0 + jnp.exp(-jnp.abs(x)))


def _bf16_bits(xbits):
    return (xbits + jnp.int32(0x7FFF) + ((xbits >> 16) & 1)) >> 16


def _g_table_body(atom_ref, wnbr_ref, gp_ref):
    g = jnp.dot(atom_ref[...], wnbr_ref[...],
                preferred_element_type=jnp.float32)
    xb = lax.bitcast_convert_type(g, jnp.int32)
    lo = _bf16_bits(xb[:, :F_ATOM]) & jnp.int32(0xFFFF)
    hi = _bf16_bits(xb[:, F_ATOM:]) << 16
    gp_ref[...] = hi | lo


def _g_table(atom_fea, wnbr):
    bp = 2000
    return pl.pallas_call(
        _g_table_body,
        grid=(N // bp,),
        in_specs=[
            pl.BlockSpec((bp, F_ATOM), lambda i: (i, 0)),
            pl.BlockSpec((F_ATOM, F_OUT), lambda i: (0, 0)),
        ],
        out_specs=pl.BlockSpec((bp, F_ATOM), lambda i: (i, 0)),
        out_shape=jax.ShapeDtypeStruct((N, F_ATOM), jnp.int32),
    )(atom_fea, wnbr)


def _perm_idx(idx_flat):
    s = jnp.arange(EDGES, dtype=jnp.int32)
    c = s // _C
    q = s % _C
    h = q // (_C // 2)
    p = (_C // 2) * c + q % (_C // 2)
    blk = p // _HB
    r = p % _HB
    e = 2 * _HB * blk + _HB * h + r
    return idx_flat[e]


def _sc_gather(gp_table, idx_perm):
    info = plsc.get_sparse_core_info()
    nc, ns = info.num_cores, info.num_subcores
    nw = nc * ns
    bpw = EDGES // nw
    nchunk = bpw // _C
    mesh = plsc.VectorSubcoreMesh(core_axis_name="c", subcore_axis_name="s")

    @functools.partial(
        pl.kernel,
        out_type=jax.ShapeDtypeStruct((EDGES // 2, F_OUT), jnp.int32),
        mesh=mesh,
        scratch_types=[
            pltpu.VMEM((_C,), jnp.int32),
            pltpu.VMEM((_C, F_ATOM), jnp.int32),
            pltpu.SemaphoreType.DMA,
        ],
        compiler_params=pltpu.CompilerParams(use_tc_tiling_on_sc=False),
    )
    def gather_kernel(table_hbm, idx_hbm, out_hbm, idx_v, rows_v, sem):
        wid = lax.axis_index("s") * nc + lax.axis_index("c")
        base = wid * bpw
        for i in range(nchunk):
            off = base + i * _C
            pltpu.sync_copy(idx_hbm.at[pl.ds(off, _C)], idx_v)
            pltpu.async_copy(table_hbm.at[idx_v], rows_v, sem).wait()
            orow = off // 2
            pltpu.sync_copy(
                rows_v.at[pl.ds(0, _C // 2)],
                out_hbm.at[pl.ds(orow, _C // 2), pl.ds(0, F_ATOM)])
            pltpu.sync_copy(
                rows_v.at[pl.ds(_C // 2, _C // 2)],
                out_hbm.at[pl.ds(orow, _C // 2), pl.ds(F_ATOM, F_ATOM)])

    return gather_kernel(gp_table, idx_perm)


def _unpack_gp(gath_ref):
    x = gath_ref[...]
    lo = lax.bitcast_convert_type(x << 16, jnp.float32)
    hi = lax.bitcast_convert_type(x & jnp.int32(-65536), jnp.float32)
    a = jnp.concatenate([lo[:, :F_ATOM], hi[:, :F_ATOM]], axis=1)
    b = jnp.concatenate([lo[:, F_ATOM:], hi[:, F_ATOM:]], axis=1)
    return jnp.concatenate([a, b], axis=0)


def _edge_gated(gath_ref, nbr_ref, atom_ref, wself_ref, wedge_ref, bias_ref):
    r = _B * M
    s = jnp.dot(atom_ref[...], wself_ref[...],
                preferred_element_type=jnp.float32) + bias_ref[...]
    gp = _unpack_gp(gath_ref)
    e = jnp.dot(nbr_ref[...].reshape(r, F_NBR), wedge_ref[...],
                preferred_element_type=jnp.float32)
    return (gp + e).reshape(_B, M, F_OUT) + s[:, None, :]


def _k1_body(gath_ref, nbr_ref, atom_ref, wself_ref, wedge_ref,
             bias_ref, s1_ref, s2_ref, xh_ref):
    gated = _edge_gated(gath_ref, nbr_ref, atom_ref, wself_ref,
                        wedge_ref, bias_ref)
    g2 = gated.reshape(_B * M, F_OUT)
    xh_ref[...] = g2.astype(jnp.bfloat16)

    @pl.when(pl.program_id(0) == 0)
    def _():
        s1_ref[...] = jnp.zeros_like(s1_ref)
        s2_ref[...] = jnp.zeros_like(s2_ref)

    s1_ref[...] += jnp.sum(g2, axis=0, keepdims=True)
    s2_ref[...] += jnp.sum(g2 * g2, axis=0, keepdims=True)


def _k2_body(xh_in_ref, s1_ref, s2_ref, g1_ref, b1_ref,
             ns_ref, t1_ref, t2_ref):
    mean = s1_ref[...] / EDGES
    var = s2_ref[...] / EDGES - mean * mean
    scale = g1_ref[...] * lax.rsqrt(var + EPS)
    shift = b1_ref[...] - mean * scale

    gated = xh_in_ref[...].astype(jnp.float32).reshape(_B, M, F_OUT)
    xh = gated * scale.reshape(1, 1, F_OUT) + shift.reshape(1, 1, F_OUT)
    filt = _sigmoid(xh[..., :F_ATOM])
    core = _softplus(xh[..., F_ATOM:])
    ns = jnp.sum(filt * core, axis=1)
    ns_ref[...] = ns

    @pl.when(pl.program_id(0) == 0)
    def _():
        t1_ref[...] = jnp.zeros_like(t1_ref)
        t2_ref[...] = jnp.zeros_like(t2_ref)

    t1_ref[...] += jnp.sum(ns, axis=0, keepdims=True)
    t2_ref[...] += jnp.sum(ns * ns, axis=0, keepdims=True)


def _k3_body(atom_ref, ns_ref, t1_ref, t2_ref, g2_ref, b2_ref, out_ref):
    mean = t1_ref[...] / N
    var = t2_ref[...] / N - mean * mean
    scale = g2_ref[...] * lax.rsqrt(var + EPS)
    shift = b2_ref[...] - mean * scale
    out_ref[...] = _softplus(atom_ref[...] + ns_ref[...] * scale + shift)


def kernel(atom_fea, nbr_fea, nbr_fea_idx, W_full, b_full,
           bn1_gamma, bn1_beta, bn2_gamma, bn2_beta):
    idx_flat = nbr_fea_idx.reshape(-1).astype(jnp.int32)
    wself = W_full[:, :F_ATOM].T
    wnbr = W_full[:, F_ATOM:2 * F_ATOM].T
    wedge = W_full[:, 2 * F_ATOM:].T
    gp_table = _g_table(atom_fea, wnbr)
    gathered = _sc_gather(gp_table, _perm_idx(idx_flat))
    bias = b_full.reshape(1, F_OUT)
    g1 = bn1_gamma.reshape(1, F_OUT)
    b1 = bn1_beta.reshape(1, F_OUT)
    g2 = bn2_gamma.reshape(1, F_ATOM)
    b2 = bn2_beta.reshape(1, F_ATOM)

    grid = (N // _B,)
    const2 = lambda s: pl.BlockSpec(s, lambda i: (0, 0))
    edge_specs = [
        pl.BlockSpec((_HB, F_OUT), lambda i: (i, 0)),
        pl.BlockSpec((_B, M, F_NBR), lambda i: (i, 0, 0)),
        pl.BlockSpec((_B, F_ATOM), lambda i: (i, 0)),
        const2((F_ATOM, F_OUT)),
        const2((F_NBR, F_OUT)),
        const2((1, F_OUT)),
    ]

    s1, s2, xh = pl.pallas_call(
        _k1_body,
        grid=grid,
        in_specs=edge_specs,
        out_specs=[const2((1, F_OUT)), const2((1, F_OUT)),
                   pl.BlockSpec((_B * M, F_OUT), lambda i: (i, 0))],
        out_shape=[jax.ShapeDtypeStruct((1, F_OUT), jnp.float32)] * 2
        + [jax.ShapeDtypeStruct((EDGES, F_OUT), jnp.bfloat16)],
        compiler_params=pltpu.CompilerParams(
            dimension_semantics=("arbitrary",)),
    )(gathered, nbr_fea, atom_fea, wself, wedge, bias)

    ns, t1, t2 = pl.pallas_call(
        _k2_body,
        grid=grid,
        in_specs=[pl.BlockSpec((_B * M, F_OUT), lambda i: (i, 0))]
        + [const2((1, F_OUT))] * 4,
        out_specs=[
            pl.BlockSpec((_B, F_ATOM), lambda i: (i, 0)),
            const2((1, F_ATOM)),
            const2((1, F_ATOM)),
        ],
        out_shape=[
            jax.ShapeDtypeStruct((N, F_ATOM), jnp.float32),
            jax.ShapeDtypeStruct((1, F_ATOM), jnp.float32),
            jax.ShapeDtypeStruct((1, F_ATOM), jnp.float32),
        ],
        compiler_params=pltpu.CompilerParams(
            dimension_semantics=("arbitrary",)),
    )(xh, s1, s2, g1, b1)

    b3 = 2000
    out = pl.pallas_call(
        _k3_body,
        grid=(N // b3,),
        in_specs=[
            pl.BlockSpec((b3, F_ATOM), lambda i: (i, 0)),
            pl.BlockSpec((b3, F_ATOM), lambda i: (i, 0)),
            const2((1, F_ATOM)),
            const2((1, F_ATOM)),
            const2((1, F_ATOM)),
            const2((1, F_ATOM)),
        ],
        out_specs=pl.BlockSpec((b3, F_ATOM), lambda i: (i, 0)),
        out_shape=jax.ShapeDtypeStruct((N, F_ATOM), jnp.float32),
        compiler_params=pltpu.CompilerParams(
            dimension_semantics=("parallel",)),
    )(atom_fea, ns, t1, t2, g2, b2)

    return out


# speedup vs baseline: 1.1250x; 1.1250x over previous
import functools

import jax
import jax.numpy as jnp
from jax import lax
from jax.experimental import pallas as pl
from jax.experimental.pallas import tpu as pltpu
from jax.experimental.pallas import tpu_sc as plsc

N = 50000
M = 16
F_ATOM = 64
F_NBR = 41
F_OUT = 128
EDGES = N * M
EPS = 1e-5

_B = 400
_C = 800
_HB = _B * M // 2


def _sigmoid(x):
    return 1.0 / (1.0 + jnp.exp(-x))


def _softplus(x):
    return jnp.maximum(x, 0.0) + jnp.log(1.0 + jnp.exp(-jnp.abs(x)))


def _bf16_bits(xbits):
    return (xbits + jnp.int32(0x7FFF) + ((xbits >> 16) & 1)) >> 16


def _g_table_body(atom_ref, wnbr_ref, gp_ref):
    g = jnp.dot(atom_ref[...], wnbr_ref[...],
                preferred_element_type=jnp.float32)
    xb = lax.bitcast_convert_type(g, jnp.int32)
    lo = _bf16_bits(xb[:, :F_ATOM]) & jnp.int32(0xFFFF)
    hi = _bf16_bits(xb[:, F_ATOM:]) << 16
    gp_ref[...] = hi | lo


def _g_table(atom_fea, wnbr):
    bp = 2000
    return pl.pallas_call(
        _g_table_body,
        grid=(N // bp,),
        in_specs=[
            pl.BlockSpec((bp, F_ATOM), lambda i: (i, 0)),
            pl.BlockSpec((F_ATOM, F_OUT), lambda i: (0, 0)),
        ],
        out_specs=pl.BlockSpec((bp, F_ATOM), lambda i: (i, 0)),
        out_shape=jax.ShapeDtypeStruct((N, F_ATOM), jnp.int32),
    )(atom_fea, wnbr)


def _sc_gather(gp_table, idx_flat):
    info = plsc.get_sparse_core_info()
    nc, ns = info.num_cores, info.num_subcores
    nw = nc * ns
    nchunks = EDGES // _C
    hc = _C // 2
    per_blk = _HB // hc
    mesh = plsc.VectorSubcoreMesh(core_axis_name="c", subcore_axis_name="s")

    @functools.partial(
        pl.kernel,
        out_type=jax.ShapeDtypeStruct((EDGES // 2, F_OUT), jnp.int32),
        mesh=mesh,
        scratch_types=[
            pltpu.VMEM((_C,), jnp.int32),
            pltpu.VMEM((_C, F_ATOM), jnp.int32),
            pltpu.SemaphoreType.DMA,
        ],
        compiler_params=pltpu.CompilerParams(use_tc_tiling_on_sc=False),
    )
    def gather_kernel(table_hbm, idx_hbm, out_hbm, idx_v, rows_v, sem):
        wid = lax.axis_index("s") * nc + lax.axis_index("c")
        for j in range((nchunks + nw - 1) // nw):
            c = wid + j * nw

            @pl.when(c < nchunks)
            def _():
                ea = (c // per_blk) * 2 * _HB + (c % per_blk) * hc
                pltpu.sync_copy(idx_hbm.at[pl.ds(ea, hc)],
                                idx_v.at[pl.ds(0, hc)])
                pltpu.sync_copy(idx_hbm.at[pl.ds(ea + _HB, hc)],
                                idx_v.at[pl.ds(hc, hc)])
                pltpu.async_copy(table_hbm.at[idx_v], rows_v, sem).wait()
                orow = c * hc
                pltpu.sync_copy(
                    rows_v.at[pl.ds(0, hc)],
                    out_hbm.at[pl.ds(orow, hc), pl.ds(0, F_ATOM)])
                pltpu.sync_copy(
                    rows_v.at[pl.ds(hc, hc)],
                    out_hbm.at[pl.ds(orow, hc), pl.ds(F_ATOM, F_ATOM)])

    return gather_kernel(gp_table, idx_flat)


def _unpack_gp(gath_ref):
    x = gath_ref[...]
    lo = lax.bitcast_convert_type(x << 16, jnp.float32)
    hi = lax.bitcast_convert_type(x & jnp.int32(-65536), jnp.float32)
    a = jnp.concatenate([lo[:, :F_ATOM], hi[:, :F_ATOM]], axis=1)
    b = jnp.concatenate([lo[:, F_ATOM:], hi[:, F_ATOM:]], axis=1)
    return jnp.concatenate([a, b], axis=0)


def _edge_gated(gath_ref, nbr_ref, atom_ref, wself_ref, wedge_ref, bias_ref):
    r = _B * M
    s = jnp.dot(atom_ref[...], wself_ref[...],
                preferred_element_type=jnp.float32) + bias_ref[...]
    gp = _unpack_gp(gath_ref)
    e = jnp.dot(nbr_ref[...].reshape(r, F_NBR), wedge_ref[...],
                preferred_element_type=jnp.float32)
    return (gp + e).reshape(_B, M, F_OUT) + s[:, None, :]


def _k1_body(gath_ref, nbr_ref, atom_ref, wself_ref, wedge_ref,
             bias_ref, s1_ref, s2_ref, xh_ref):
    gated = _edge_gated(gath_ref, nbr_ref, atom_ref, wself_ref,
                        wedge_ref, bias_ref)
    g2 = gated.reshape(_B * M, F_OUT)
    xh_ref[...] = g2.astype(jnp.bfloat16)

    @pl.when(pl.program_id(0) == 0)
    def _():
        s1_ref[...] = jnp.zeros_like(s1_ref)
        s2_ref[...] = jnp.zeros_like(s2_ref)

    s1_ref[...] += jnp.sum(g2, axis=0, keepdims=True)
    s2_ref[...] += jnp.sum(g2 * g2, axis=0, keepdims=True)


def _k2_body(xh_in_ref, s1_ref, s2_ref, g1_ref, b1_ref,
             ns_ref, t1_ref, t2_ref):
    mean = s1_ref[...] / EDGES
    var = s2_ref[...] / EDGES - mean * mean
    scale = g1_ref[...] * lax.rsqrt(var + EPS)
    shift = b1_ref[...] - mean * scale

    gated = xh_in_ref[...].astype(jnp.float32).reshape(_B, M, F_OUT)
    xh = gated * scale.reshape(1, 1, F_OUT) + shift.reshape(1, 1, F_OUT)
    filt = _sigmoid(xh[..., :F_ATOM])
    core = _softplus(xh[..., F_ATOM:])
    ns = jnp.sum(filt * core, axis=1)
    ns_ref[...] = ns

    @pl.when(pl.program_id(0) == 0)
    def _():
        t1_ref[...] = jnp.zeros_like(t1_ref)
        t2_ref[...] = jnp.zeros_like(t2_ref)

    t1_ref[...] += jnp.sum(ns, axis=0, keepdims=True)
    t2_ref[...] += jnp.sum(ns * ns, axis=0, keepdims=True)


def _k3_body(atom_ref, ns_ref, t1_ref, t2_ref, g2_ref, b2_ref, out_ref):
    mean = t1_ref[...] / N
    var = t2_ref[...] / N - mean * mean
    scale = g2_ref[...] * lax.rsqrt(var + EPS)
    shift = b2_ref[...] - mean * scale
    out_ref[...] = _softplus(atom_ref[...] + ns_ref[...] * scale + shift)


def kernel(atom_fea, nbr_fea, nbr_fea_idx, W_full, b_full,
           bn1_gamma, bn1_beta, bn2_gamma, bn2_beta):
    idx_flat = nbr_fea_idx.reshape(-1).astype(jnp.int32)
    wself = W_full[:, :F_ATOM].T
    wnbr = W_full[:, F_ATOM:2 * F_ATOM].T
    wedge = W_full[:, 2 * F_ATOM:].T
    gp_table = _g_table(atom_fea, wnbr)
    gathered = _sc_gather(gp_table, idx_flat)
    bias = b_full.reshape(1, F_OUT)
    g1 = bn1_gamma.reshape(1, F_OUT)
    b1 = bn1_beta.reshape(1, F_OUT)
    g2 = bn2_gamma.reshape(1, F_ATOM)
    b2 = bn2_beta.reshape(1, F_ATOM)

    grid = (N // _B,)
    const2 = lambda s: pl.BlockSpec(s, lambda i: (0, 0))
    edge_specs = [
        pl.BlockSpec((_HB, F_OUT), lambda i: (i, 0)),
        pl.BlockSpec((_B, M, F_NBR), lambda i: (i, 0, 0)),
        pl.BlockSpec((_B, F_ATOM), lambda i: (i, 0)),
        const2((F_ATOM, F_OUT)),
        const2((F_NBR, F_OUT)),
        const2((1, F_OUT)),
    ]

    s1, s2, xh = pl.pallas_call(
        _k1_body,
        grid=grid,
        in_specs=edge_specs,
        out_specs=[const2((1, F_OUT)), const2((1, F_OUT)),
                   pl.BlockSpec((_B * M, F_OUT), lambda i: (i, 0))],
        out_shape=[jax.ShapeDtypeStruct((1, F_OUT), jnp.float32)] * 2
        + [jax.ShapeDtypeStruct((EDGES, F_OUT), jnp.bfloat16)],
        compiler_params=pltpu.CompilerParams(
            dimension_semantics=("arbitrary",)),
    )(gathered, nbr_fea, atom_fea, wself, wedge, bias)

    ns, t1, t2 = pl.pallas_call(
        _k2_body,
        grid=grid,
        in_specs=[pl.BlockSpec((_B * M, F_OUT), lambda i: (i, 0))]
        + [const2((1, F_OUT))] * 4,
        out_specs=[
            pl.BlockSpec((_B, F_ATOM), lambda i: (i, 0)),
            const2((1, F_ATOM)),
            const2((1, F_ATOM)),
        ],
        out_shape=[
            jax.ShapeDtypeStruct((N, F_ATOM), jnp.float32),
            jax.ShapeDtypeStruct((1, F_ATOM), jnp.float32),
            jax.ShapeDtypeStruct((1, F_ATOM), jnp.float32),
        ],
        compiler_params=pltpu.CompilerParams(
            dimension_semantics=("arbitrary",)),
    )(xh, s1, s2, g1, b1)

    b3 = 2000
    out = pl.pallas_call(
        _k3_body,
        grid=(N // b3,),
        in_specs=[
            pl.BlockSpec((b3, F_ATOM), lambda i: (i, 0)),
            pl.BlockSpec((b3, F_ATOM), lambda i: (i, 0)),
            const2((1, F_ATOM)),
            const2((1, F_ATOM)),
            const2((1, F_ATOM)),
            const2((1, F_ATOM)),
        ],
        out_specs=pl.BlockSpec((b3, F_ATOM), lambda i: (i, 0)),
        out_shape=jax.ShapeDtypeStruct((N, F_ATOM), jnp.float32),
        compiler_params=pltpu.CompilerParams(
            dimension_semantics=("parallel",)),
    )(atom_fea, ns, t1, t2, g2, b2)

    return out


# speedup vs baseline: 1.1354x; 1.0092x over previous
import functools

import jax
import jax.numpy as jnp
from jax import lax
from jax.experimental import pallas as pl
from jax.experimental.pallas import tpu as pltpu
from jax.experimental.pallas import tpu_sc as plsc

N = 50000
M = 16
F_ATOM = 64
F_NBR = 41
F_OUT = 128
EDGES = N * M
EPS = 1e-5

_B = 400
_C = 800
_HB = _B * M // 2


def _sigmoid(x):
    return 0.5 * jnp.tanh(0.5 * x) + 0.5


def _softplus(x):
    return jnp.maximum(x, 0.0) + jnp.log(1.0 + jnp.exp(-jnp.abs(x)))


def _bf16_bits(xbits):
    return (xbits + jnp.int32(0x7FFF) + ((xbits >> 16) & 1)) >> 16


def _g_table_body(atom_ref, wnbr_ref, gp_ref):
    g = jnp.dot(atom_ref[...], wnbr_ref[...],
                preferred_element_type=jnp.float32)
    xb = lax.bitcast_convert_type(g, jnp.int32)
    lo = _bf16_bits(xb[:, :F_ATOM]) & jnp.int32(0xFFFF)
    hi = _bf16_bits(xb[:, F_ATOM:]) << 16
    gp_ref[...] = hi | lo


def _g_table(atom_fea, wnbr):
    bp = 2000
    return pl.pallas_call(
        _g_table_body,
        grid=(N // bp,),
        in_specs=[
            pl.BlockSpec((bp, F_ATOM), lambda i: (i, 0)),
            pl.BlockSpec((F_ATOM, F_OUT), lambda i: (0, 0)),
        ],
        out_specs=pl.BlockSpec((bp, F_ATOM), lambda i: (i, 0)),
        out_shape=jax.ShapeDtypeStruct((N, F_ATOM), jnp.int32),
    )(atom_fea, wnbr)


def _sc_gather(gp_table, idx_flat):
    info = plsc.get_sparse_core_info()
    nc, ns = info.num_cores, info.num_subcores
    nw = nc * ns
    nchunks = EDGES // _C
    hc = _C // 2
    per_blk = _HB // hc
    mesh = plsc.VectorSubcoreMesh(core_axis_name="c", subcore_axis_name="s")

    @functools.partial(
        pl.kernel,
        out_type=jax.ShapeDtypeStruct((EDGES // 2, F_OUT), jnp.int32),
        mesh=mesh,
        scratch_types=[
            pltpu.VMEM((_C,), jnp.int32),
            pltpu.VMEM((_C, F_ATOM), jnp.int32),
            pltpu.SemaphoreType.DMA,
        ],
        compiler_params=pltpu.CompilerParams(use_tc_tiling_on_sc=False),
    )
    def gather_kernel(table_hbm, idx_hbm, out_hbm, idx_v, rows_v, sem):
        wid = lax.axis_index("s") * nc + lax.axis_index("c")
        for j in range((nchunks + nw - 1) // nw):
            c = wid + j * nw

            @pl.when(c < nchunks)
            def _():
                ea = (c // per_blk) * 2 * _HB + (c % per_blk) * hc
                pltpu.sync_copy(idx_hbm.at[pl.ds(ea, hc)],
                                idx_v.at[pl.ds(0, hc)])
                pltpu.sync_copy(idx_hbm.at[pl.ds(ea + _HB, hc)],
                                idx_v.at[pl.ds(hc, hc)])
                pltpu.async_copy(table_hbm.at[idx_v], rows_v, sem).wait()
                orow = c * hc
                pltpu.sync_copy(
                    rows_v.at[pl.ds(0, hc)],
                    out_hbm.at[pl.ds(orow, hc), pl.ds(0, F_ATOM)])
                pltpu.sync_copy(
                    rows_v.at[pl.ds(hc, hc)],
                    out_hbm.at[pl.ds(orow, hc), pl.ds(F_ATOM, F_ATOM)])

    return gather_kernel(gp_table, idx_flat)


def _unpack_gp(gath_ref):
    x = gath_ref[...]
    lo = lax.bitcast_convert_type(x << 16, jnp.float32)
    hi = lax.bitcast_convert_type(x & jnp.int32(-65536), jnp.float32)
    a = jnp.concatenate([lo[:, :F_ATOM], hi[:, :F_ATOM]], axis=1)
    b = jnp.concatenate([lo[:, F_ATOM:], hi[:, F_ATOM:]], axis=1)
    return jnp.concatenate([a, b], axis=0)


def _edge_gated(gath_ref, nbr_ref, atom_ref, wself_ref, wedge_ref, bias_ref):
    r = _B * M
    s = jnp.dot(atom_ref[...].astype(jnp.bfloat16),
                wself_ref[...].astype(jnp.bfloat16),
                preferred_element_type=jnp.float32) + bias_ref[...]
    gp = _unpack_gp(gath_ref)
    e = jnp.dot(nbr_ref[...].reshape(r, F_NBR).astype(jnp.bfloat16),
                wedge_ref[...].astype(jnp.bfloat16),
                preferred_element_type=jnp.float32)
    return (gp + e).reshape(_B, M, F_OUT) + s[:, None, :]


def _k1_body(gath_ref, nbr_ref, atom_ref, wself_ref, wedge_ref,
             bias_ref, s1_ref, s2_ref, xh_ref):
    gated = _edge_gated(gath_ref, nbr_ref, atom_ref, wself_ref,
                        wedge_ref, bias_ref)
    g2 = gated.reshape(_B * M, F_OUT)
    xh_ref[...] = g2.astype(jnp.bfloat16)

    @pl.when(pl.program_id(0) == 0)
    def _():
        s1_ref[...] = jnp.zeros_like(s1_ref)
        s2_ref[...] = jnp.zeros_like(s2_ref)

    s1_ref[...] += jnp.sum(g2, axis=0, keepdims=True)
    s2_ref[...] += jnp.sum(g2 * g2, axis=0, keepdims=True)


def _k2_body(xh_in_ref, s1_ref, s2_ref, g1_ref, b1_ref,
             ns_ref, t1_ref, t2_ref):
    mean = s1_ref[...] / EDGES
    var = s2_ref[...] / EDGES - mean * mean
    scale = g1_ref[...] * lax.rsqrt(var + EPS)
    shift = b1_ref[...] - mean * scale

    gated = xh_in_ref[...].astype(jnp.float32).reshape(_B, M, F_OUT)
    xh = gated * scale.reshape(1, 1, F_OUT) + shift.reshape(1, 1, F_OUT)
    filt = _sigmoid(xh[..., :F_ATOM])
    core = _softplus(xh[..., F_ATOM:])
    ns = jnp.sum(filt * core, axis=1)
    ns_ref[...] = ns

    @pl.when(pl.program_id(0) == 0)
    def _():
        t1_ref[...] = jnp.zeros_like(t1_ref)
        t2_ref[...] = jnp.zeros_like(t2_ref)

    t1_ref[...] += jnp.sum(ns, axis=0, keepdims=True)
    t2_ref[...] += jnp.sum(ns * ns, axis=0, keepdims=True)


def _k3_body(atom_ref, ns_ref, t1_ref, t2_ref, g2_ref, b2_ref, out_ref):
    mean = t1_ref[...] / N
    var = t2_ref[...] / N - mean * mean
    scale = g2_ref[...] * lax.rsqrt(var + EPS)
    shift = b2_ref[...] - mean * scale
    out_ref[...] = _softplus(atom_ref[...] + ns_ref[...] * scale + shift)


def kernel(atom_fea, nbr_fea, nbr_fea_idx, W_full, b_full,
           bn1_gamma, bn1_beta, bn2_gamma, bn2_beta):
    idx_flat = nbr_fea_idx.reshape(-1).astype(jnp.int32)
    wself = W_full[:, :F_ATOM].T
    wnbr = W_full[:, F_ATOM:2 * F_ATOM].T
    wedge = W_full[:, 2 * F_ATOM:].T
    gp_table = _g_table(atom_fea, wnbr)
    gathered = _sc_gather(gp_table, idx_flat)
    bias = b_full.reshape(1, F_OUT)
    g1 = bn1_gamma.reshape(1, F_OUT)
    b1 = bn1_beta.reshape(1, F_OUT)
    g2 = bn2_gamma.reshape(1, F_ATOM)
    b2 = bn2_beta.reshape(1, F_ATOM)

    grid = (N // _B,)
    const2 = lambda s: pl.BlockSpec(s, lambda i: (0, 0))
    edge_specs = [
        pl.BlockSpec((_HB, F_OUT), lambda i: (i, 0)),
        pl.BlockSpec((_B, M, F_NBR), lambda i: (i, 0, 0)),
        pl.BlockSpec((_B, F_ATOM), lambda i: (i, 0)),
        const2((F_ATOM, F_OUT)),
        const2((F_NBR, F_OUT)),
        const2((1, F_OUT)),
    ]

    s1, s2, xh = pl.pallas_call(
        _k1_body,
        grid=grid,
        in_specs=edge_specs,
        out_specs=[const2((1, F_OUT)), const2((1, F_OUT)),
                   pl.BlockSpec((_B * M, F_OUT), lambda i: (i, 0))],
        out_shape=[jax.ShapeDtypeStruct((1, F_OUT), jnp.float32)] * 2
        + [jax.ShapeDtypeStruct((EDGES, F_OUT), jnp.bfloat16)],
        compiler_params=pltpu.CompilerParams(
            dimension_semantics=("arbitrary",)),
    )(gathered, nbr_fea, atom_fea, wself, wedge, bias)

    ns, t1, t2 = pl.pallas_call(
        _k2_body,
        grid=grid,
        in_specs=[pl.BlockSpec((_B * M, F_OUT), lambda i: (i, 0))]
        + [const2((1, F_OUT))] * 4,
        out_specs=[
            pl.BlockSpec((_B, F_ATOM), lambda i: (i, 0)),
            const2((1, F_ATOM)),
            const2((1, F_ATOM)),
        ],
        out_shape=[
            jax.ShapeDtypeStruct((N, F_ATOM), jnp.float32),
            jax.ShapeDtypeStruct((1, F_ATOM), jnp.float32),
            jax.ShapeDtypeStruct((1, F_ATOM), jnp.float32),
        ],
        compiler_params=pltpu.CompilerParams(
            dimension_semantics=("arbitrary",)),
    )(xh, s1, s2, g1, b1)

    b3 = 2000
    out = pl.pallas_call(
        _k3_body,
        grid=(N // b3,),
        in_specs=[
            pl.BlockSpec((b3, F_ATOM), lambda i: (i, 0)),
            pl.BlockSpec((b3, F_ATOM), lambda i: (i, 0)),
            const2((1, F_ATOM)),
            const2((1, F_ATOM)),
            const2((1, F_ATOM)),
            const2((1, F_ATOM)),
        ],
        out_specs=pl.BlockSpec((b3, F_ATOM), lambda i: (i, 0)),
        out_shape=jax.ShapeDtypeStruct((N, F_ATOM), jnp.float32),
        compiler_params=pltpu.CompilerParams(
            dimension_semantics=("parallel",)),
    )(atom_fea, ns, t1, t2, g2, b2)

    return out
